# Initial kernel scaffold; baseline (speedup 1.0000x reference)
import jax
import jax.numpy as jnp
from jax.experimental import pallas as pl


def kernel(x, table, rm, rv, gamma, beta, W1, b1, W2, b2, W3, b3):
    raise NotImplementedError("write your pallas kernel here")


# speedup vs baseline: 21.0683x; 21.0683x over previous
import functools

import jax
import jax.numpy as jnp
from jax import lax
from jax.experimental import pallas as pl
from jax.experimental.pallas import tpu as pltpu
from jax.experimental.pallas import tpu_sc as plsc

B = 16384
F = 100
EMB = 16
D_IN = F * EMB
EPS = 1e-5

NC, NS = 2, 16
NW = NC * NS
N_ROWS = B * F
IDX_W = 128
IDX_ROWS = N_ROWS // IDX_W
IROWS_PER_W = IDX_ROWS // NW
ROWS_PER_W = N_ROWS // NW
GPC = 8
CHUNK = GPC * IDX_W
NCHUNK = ROWS_PER_W // CHUNK


def _sc_gather(table, idx2d):
    mesh = plsc.VectorSubcoreMesh(core_axis_name="c", subcore_axis_name="s")

    @functools.partial(
        pl.kernel,
        out_type=jax.ShapeDtypeStruct((N_ROWS, EMB), jnp.float32),
        mesh=mesh,
        scratch_types=[
            pltpu.VMEM((IROWS_PER_W, IDX_W), jnp.int32),
            pltpu.VMEM((CHUNK, EMB), jnp.float32),
            pltpu.SemaphoreType.DMA,
        ],
        compiler_params=pltpu.CompilerParams(use_tc_tiling_on_sc=False),
    )
    def gather_kernel(table_hbm, idx_hbm, out_hbm, idx_v, rows_v, sem):
        wid = lax.axis_index("s") * NC + lax.axis_index("c")
        pltpu.sync_copy(idx_hbm.at[pl.ds(wid * IROWS_PER_W, IROWS_PER_W)], idx_v)
        row_base = wid * ROWS_PER_W

        @pl.loop(0, NCHUNK)
        def _chunk(c):
            handles = []
            for j in range(GPC):
                handles.append(pltpu.async_copy(
                    table_hbm.at[idx_v.at[c * GPC + j]],
                    rows_v.at[pl.ds(j * IDX_W, IDX_W)],
                    sem))
            for h in handles:
                h.wait()
            pltpu.sync_copy(rows_v, out_hbm.at[pl.ds(row_base + c * CHUNK, CHUNK)])

    return gather_kernel(table, idx2d)


def _mlp_body(emb, s, t, w1, b1, w2, b2, w3, b3, out):
    a = (emb[...] * s[...] + t[...]).astype(jnp.bfloat16)
    h = jnp.dot(a, w1[...], preferred_element_type=jnp.float32) + b1[...]
    h = jnp.maximum(h, 0.0).astype(jnp.bfloat16)
    h = jnp.dot(h, w2[...], preferred_element_type=jnp.float32) + b2[...]
    h = jnp.maximum(h, 0.0).astype(jnp.bfloat16)
    z = jnp.dot(h, w3[...], preferred_element_type=jnp.float32)
    out[...] = jax.nn.sigmoid(z[:, 0:1] + b3[...])


def _tc_mlp(emb, s, t, w1, b1, w2, b2, w3, b3, bm=512):
    grid = (B // bm,)
    return pl.pallas_call(
        _mlp_body,
        grid=grid,
        in_specs=[
            pl.BlockSpec((bm, D_IN), lambda i: (i, 0)),
            pl.BlockSpec((1, D_IN), lambda i: (0, 0)),
            pl.BlockSpec((1, D_IN), lambda i: (0, 0)),
            pl.BlockSpec((D_IN, 1024), lambda i: (0, 0)),
            pl.BlockSpec((1, 1024), lambda i: (0, 0)),
            pl.BlockSpec((1024, 512), lambda i: (0, 0)),
            pl.BlockSpec((1, 512), lambda i: (0, 0)),
            pl.BlockSpec((512, 128), lambda i: (0, 0)),
            pl.BlockSpec((1, 1), lambda i: (0, 0)),
        ],
        out_specs=pl.BlockSpec((bm, 1), lambda i: (i, 0)),
        out_shape=jax.ShapeDtypeStruct((B, 1), jnp.float32),
    )(emb, s, t, w1, b1, w2, b2, w3, b3)


def kernel(x, table, rm, rv, gamma, beta, W1, b1, W2, b2, W3, b3):
    idx2d = x.astype(jnp.int32).reshape(IDX_ROWS, IDX_W)
    emb = _sc_gather(table, idx2d).reshape(B, D_IN)
    inv = lax.rsqrt(rv + EPS)
    s = (gamma * inv).reshape(1, D_IN)
    t = (beta - rm * gamma * inv).reshape(1, D_IN)
    w1 = W1.astype(jnp.bfloat16)
    w2 = W2.astype(jnp.bfloat16)
    w3 = jnp.pad(W3, ((0, 0), (0, 127))).astype(jnp.bfloat16)
    return _tc_mlp(emb, s, t, w1, b1.reshape(1, -1), w2, b2.reshape(1, -1),
                   w3, b3.reshape(1, 1))


# speedup vs baseline: 23.5404x; 1.1173x over previous
import functools

import jax
import jax.numpy as jnp
from jax import lax
from jax.experimental import pallas as pl
from jax.experimental.pallas import tpu as pltpu
from jax.experimental.pallas import tpu_sc as plsc

B = 16384
F = 100
EMB = 16
D_IN = F * EMB
EPS = 1e-5

NC, NS = 2, 16
NW = NC * NS
N_ROWS = B * F
IDX_W = 128
IDX_ROWS = N_ROWS // IDX_W
IROWS_PER_W = IDX_ROWS // NW
ROWS_PER_W = N_ROWS // NW
GPC = 8
CHUNK = GPC * IDX_W
NCHUNK = ROWS_PER_W // CHUNK


def _sc_gather(table, idx2d):
    mesh = plsc.VectorSubcoreMesh(core_axis_name="c", subcore_axis_name="s")

    @functools.partial(
        pl.kernel,
        out_type=jax.ShapeDtypeStruct((N_ROWS, EMB), jnp.float32),
        mesh=mesh,
        scratch_types=[
            pltpu.VMEM((IROWS_PER_W, IDX_W), jnp.int32),
            pltpu.VMEM((CHUNK, EMB), jnp.float32),
            pltpu.VMEM((CHUNK, EMB), jnp.float32),
            pltpu.SemaphoreType.DMA,
            pltpu.SemaphoreType.DMA,
            pltpu.SemaphoreType.DMA,
            pltpu.SemaphoreType.DMA,
        ],
        compiler_params=pltpu.CompilerParams(use_tc_tiling_on_sc=False),
    )
    def gather_kernel(table_hbm, idx_hbm, out_hbm, idx_v,
                      rows0, rows1, gsem0, gsem1, osem0, osem1):
        wid = lax.axis_index("s") * NC + lax.axis_index("c")
        pltpu.sync_copy(idx_hbm.at[pl.ds(wid * IROWS_PER_W, IROWS_PER_W)], idx_v)
        row_base = wid * ROWS_PER_W
        rows = (rows0, rows1)
        gsem = (gsem0, gsem1)
        osem = (osem0, osem1)

        def fire(c, b):
            for j in range(GPC):
                pltpu.async_copy(table_hbm.at[idx_v.at[c * GPC + j]],
                                 rows[b].at[pl.ds(j * IDX_W, IDX_W)], gsem[b])

        def drain_g(b):
            pltpu.make_async_copy(table_hbm.at[pl.ds(0, CHUNK)],
                                  rows[b], gsem[b]).wait()

        def drain_o(b):
            pltpu.make_async_copy(rows[b], out_hbm.at[pl.ds(0, CHUNK)],
                                  osem[b]).wait()

        fire(0, 0)

        @pl.loop(0, NCHUNK // 2)
        def _pair(g):
            for b in range(2):
                c = 2 * g + b

                @pl.when(c >= 1)
                def _():
                    drain_o(1 - b)

                @pl.when(c + 1 < NCHUNK)
                def _():
                    fire(c + 1, 1 - b)

                drain_g(b)
                pltpu.async_copy(
                    rows[b], out_hbm.at[pl.ds(row_base + c * CHUNK, CHUNK)],
                    osem[b])

        drain_o(1)

    return gather_kernel(table, idx2d)


def _mlp_body(emb, s, t, w1, b1, w2, b2, w3, b3, out):
    a = (emb[...] * s[...] + t[...]).astype(jnp.bfloat16)
    h = jnp.dot(a, w1[...], preferred_element_type=jnp.float32) + b1[...]
    h = jnp.maximum(h, 0.0).astype(jnp.bfloat16)
    h = jnp.dot(h, w2[...], preferred_element_type=jnp.float32) + b2[...]
    h = jnp.maximum(h, 0.0).astype(jnp.bfloat16)
    z = jnp.dot(h, w3[...], preferred_element_type=jnp.float32)
    out[...] = jax.nn.sigmoid(z[:, 0:1] + b3[...])


def _tc_mlp(emb, s, t, w1, b1, w2, b2, w3, b3, bm=512):
    grid = (B // bm,)
    return pl.pallas_call(
        _mlp_body,
        grid=grid,
        in_specs=[
            pl.BlockSpec((bm, D_IN), lambda i: (i, 0)),
            pl.BlockSpec((1, D_IN), lambda i: (0, 0)),
            pl.BlockSpec((1, D_IN), lambda i: (0, 0)),
            pl.BlockSpec((D_IN, 1024), lambda i: (0, 0)),
            pl.BlockSpec((1, 1024), lambda i: (0, 0)),
            pl.BlockSpec((1024, 512), lambda i: (0, 0)),
            pl.BlockSpec((1, 512), lambda i: (0, 0)),
            pl.BlockSpec((512, 128), lambda i: (0, 0)),
            pl.BlockSpec((1, 1), lambda i: (0, 0)),
        ],
        out_specs=pl.BlockSpec((bm, 1), lambda i: (i, 0)),
        out_shape=jax.ShapeDtypeStruct((B, 1), jnp.float32),
    )(emb, s, t, w1, b1, w2, b2, w3, b3)


def kernel(x, table, rm, rv, gamma, beta, W1, b1, W2, b2, W3, b3):
    idx2d = x.astype(jnp.int32).reshape(IDX_ROWS, IDX_W)
    emb = _sc_gather(table, idx2d).reshape(B, D_IN)
    inv = lax.rsqrt(rv + EPS)
    s = (gamma * inv).reshape(1, D_IN)
    t = (beta - rm * gamma * inv).reshape(1, D_IN)
    w1 = W1.astype(jnp.bfloat16)
    w2 = W2.astype(jnp.bfloat16)
    w3 = jnp.pad(W3, ((0, 0), (0, 127))).astype(jnp.bfloat16)
    return _tc_mlp(emb, s, t, w1, b1.reshape(1, -1), w2, b2.reshape(1, -1),
                   w3, b3.reshape(1, 1))
